# baseline (speedup 1.0000x reference)
import functools

import jax
import jax.numpy as jnp
from jax import lax
from jax.experimental import pallas as pl
from jax.experimental.pallas import tpu as pltpu
from jax.experimental.pallas import tpu_sc as plsc

N = 10002
NP = 10240
D = 256
E = 160000
R = 8
B = 1024
ASP = 30
HALF = 128
NREL = R + 1
IDS = B * (2 + ASP)
NSUB = 16
K = 128
EP = 163840
EPT = EP // NSUB
ITERS = EPT // K
KO = 128
OPT = IDS // NSUB
OPAIRS = OPT // KO // 2
ZB = 1024
BM = 1024
GN = NP // BM


def _mm_body(x_ref, w_ref, b_ref, o_ref):
    acc = jnp.dot(x_ref[...], w_ref[...],
                  preferred_element_type=jnp.float32) + b_ref[...]
    o_ref[...] = acc.reshape(BM // 8, 8, 2 * NREL, HALF).transpose(
        0, 2, 1, 3).reshape(BM * 2 * NREL, HALF)


def _build_table(xp, w_cat, b_cat):
    y = pl.pallas_call(
        _mm_body,
        grid=(GN,),
        in_specs=[
            pl.BlockSpec((BM, D), lambda i: (i, 0)),
            pl.BlockSpec((D, NREL * D), lambda i: (0, 0)),
            pl.BlockSpec((1, NREL * D), lambda i: (0, 0)),
        ],
        out_specs=pl.BlockSpec((BM * 2 * NREL, HALF), lambda i: (i, 0)),
        out_shape=jax.ShapeDtypeStruct((NP * 2 * NREL, HALF), jnp.float32),
    )(xp, w_cat, b_cat)
    return y


def _fields(eb, i):
    t = eb[pl.ds(i * 16, 16)]
    m = eb[pl.ds(K + i * 16, 16)]
    a = eb[pl.ds(2 * K + i * 16, 16)]
    return t, m, a


def _cnt_body(epk_hbm, cnt_hbm, cnt, ebA, ebB, sidxA, sidxB, onesb, zb,
              lsemA, lsemB, ssemA, ssemB):
    s = lax.axis_index("s")

    def _z16(i, _):
        zb[pl.ds(i * 16, 16)] = jnp.zeros((16,), jnp.float32)
        return 0
    lax.fori_loop(0, ZB // 16, _z16, 0)
    for q in range(R * NP // NSUB // ZB):
        pltpu.sync_copy(zb, cnt.at[pl.ds(s * (R * NP // NSUB) + q * ZB, ZB)])
    for i in range(K // 16):
        onesb[pl.ds(i * 16, 16)] = jnp.ones((16,), jnp.float32)

    plsc.subcore_barrier()

    c = lax.axis_index("c")
    hw = ITERS // 4
    def _cpair(it2h, _):
        it2 = c * hw + it2h
        g0 = (s * ITERS + it2 * 2) * (3 * K)
        lda = pltpu.async_copy(epk_hbm.at[pl.ds(g0, 3 * K)], ebA, lsemA)
        ldb = pltpu.async_copy(epk_hbm.at[pl.ds(g0 + 3 * K, 3 * K)], ebB, lsemB)

        @pl.when(it2h > 0)
        def _():
            pltpu.make_async_copy(onesb, cnt.at[sidxA], ssemA).wait()
        lda.wait()

        def _mka(i, _):
            t, m, a = _fields(ebA, i)
            sidxA[pl.ds(i * 16, 16)] = t * NP + a
            return 0
        lax.fori_loop(0, K // 16, _mka, 0)
        pltpu.async_copy(onesb, cnt.at[sidxA], ssemA, add=True)

        @pl.when(it2h > 0)
        def _():
            pltpu.make_async_copy(onesb, cnt.at[sidxB], ssemB).wait()
        ldb.wait()

        def _mkb(i, _):
            t, m, a = _fields(ebB, i)
            sidxB[pl.ds(i * 16, 16)] = t * NP + a
            return 0
        lax.fori_loop(0, K // 16, _mkb, 0)
        pltpu.async_copy(onesb, cnt.at[sidxB], ssemB, add=True)
        return 0
    lax.fori_loop(0, hw, _cpair, 0)
    pltpu.make_async_copy(onesb, cnt.at[sidxA], ssemA).wait()
    pltpu.make_async_copy(onesb, cnt.at[sidxB], ssemB).wait()

    plsc.subcore_barrier()
    span = R * NP // NSUB
    o0 = s * span
    pltpu.sync_copy(cnt.at[pl.ds(o0, span)], cnt_hbm.at[pl.ds(c * R * NP + o0, span)])


_cnt_kernel = functools.partial(
    pl.kernel,
    out_type=jax.ShapeDtypeStruct((2 * R * NP,), jnp.float32),
    mesh=plsc.VectorSubcoreMesh(core_axis_name="c", subcore_axis_name="s"),
    scratch_types=[
        pltpu.VMEM_SHARED((R * NP,), jnp.float32),
        pltpu.VMEM((3 * K,), jnp.int32),
        pltpu.VMEM((3 * K,), jnp.int32),
        pltpu.VMEM((K,), jnp.int32),
        pltpu.VMEM((K,), jnp.int32),
        pltpu.VMEM((K,), jnp.float32),
        pltpu.VMEM((ZB,), jnp.float32),
        pltpu.SemaphoreType.DMA,
        pltpu.SemaphoreType.DMA,
        pltpu.SemaphoreType.DMA,
        pltpu.SemaphoreType.DMA,
    ],
)(_cnt_body)


def _sc_body(tbl_hbm, epk_hbm, ewp_hbm, ids_hbm, cnt_hbm, out_hbm,
             accum,
             ebA, ebB, ewA, ewB, rowsA, rowsB, gidxA, gidxB, cidxA, cidxB,
             cidx2A, cidx2B, sidxA, sidxB, cvalsA, cvalsB, cvals2A, cvals2B,
             coefA, coefB, oidxA, oidxB,
             lsemA, lsemB, gsemA, gsemB, csemA, csemB, ssemA, ssemB):
    c = lax.axis_index("c")
    s = lax.axis_index("s")

    lane16 = lax.iota(jnp.int32, 16)
    for q in range(NP // NSUB // K):
        n0 = s * (NP // NSUB) + q * K

        def _ri(i, _):
            n = n0 + i * 16 + lane16
            gidxA[pl.ds(i * 16, 16)] = (
                (n >> 3) * 144 + (c * NREL + R) * 8 + (n & 7))
            return 0
        lax.fori_loop(0, K // 16, _ri, 0)
        pltpu.async_copy(tbl_hbm.at[gidxA], rowsA, gsemA).wait()
        pltpu.sync_copy(rowsA, accum.at[pl.ds(n0, K)])

    plsc.subcore_barrier()

    def _pair(it2, _):
        g0 = (s * ITERS + it2 * 2) * (3 * K)
        e0 = (s * ITERS + it2 * 2) * K
        lda = pltpu.async_copy(epk_hbm.at[pl.ds(g0, 3 * K)], ebA, lsemA)
        ldb = pltpu.async_copy(epk_hbm.at[pl.ds(g0 + 3 * K, 3 * K)], ebB, lsemB)
        lwa = pltpu.async_copy(ewp_hbm.at[pl.ds(e0, K)], ewA, gsemA)
        lwb = pltpu.async_copy(ewp_hbm.at[pl.ds(e0 + K, K)], ewB, gsemB)

        @pl.when(it2 > 0)
        def _():
            pltpu.make_async_copy(rowsA, accum.at[sidxA], ssemA).wait()
        lda.wait()
        lwa.wait()

        def _mka(i, _):
            t, m, a = _fields(ebA, i)
            gidxA[pl.ds(i * 16, 16)] = (
                (m >> 3) * 144 + (c * NREL + t) * 8 + (m & 7))
            fi = t * NP + a
            cidxA[pl.ds(i * 16, 16)] = fi
            cidx2A[pl.ds(i * 16, 16)] = fi + R * NP
            sidxA[pl.ds(i * 16, 16)] = a
            return 0
        lax.fori_loop(0, K // 16, _mka, 0)
        gca = pltpu.async_copy(tbl_hbm.at[gidxA], rowsA, gsemA)
        cca = pltpu.async_copy(cnt_hbm.at[cidxA], cvalsA, csemA)
        cca2 = pltpu.async_copy(cnt_hbm.at[cidx2A], cvals2A, csemA)

        @pl.when(it2 > 0)
        def _():
            pltpu.make_async_copy(rowsB, accum.at[sidxB], ssemB).wait()
        ldb.wait()
        lwb.wait()

        def _mkb(i, _):
            t, m, a = _fields(ebB, i)
            gidxB[pl.ds(i * 16, 16)] = (
                (m >> 3) * 144 + (c * NREL + t) * 8 + (m & 7))
            fi = t * NP + a
            cidxB[pl.ds(i * 16, 16)] = fi
            cidx2B[pl.ds(i * 16, 16)] = fi + R * NP
            sidxB[pl.ds(i * 16, 16)] = a
            return 0
        lax.fori_loop(0, K // 16, _mkb, 0)
        gcb = pltpu.async_copy(tbl_hbm.at[gidxB], rowsB, gsemB)
        ccb = pltpu.async_copy(cnt_hbm.at[cidxB], cvalsB, csemB)
        ccb2 = pltpu.async_copy(cnt_hbm.at[cidx2B], cvals2B, csemB)

        cca.wait()
        cca2.wait()

        def _cfa(i, _):
            w = ewA[pl.ds(i * 16, 16)]
            cv = cvalsA[pl.ds(i * 16, 16)] + cvals2A[pl.ds(i * 16, 16)]
            coefA[pl.ds(i * 16, 16)] = w / jnp.maximum(cv, 1.0)
            return 0
        lax.fori_loop(0, K // 16, _cfa, 0)
        gca.wait()

        def _sca(i, _):
            cv = coefA[pl.ds(i * 16, 16)]
            for jj in range(16):
                cj = jnp.full((16,), cv[jj], jnp.float32)
                for cc in range(HALF // 16):
                    rowsA[i * 16 + jj, pl.ds(cc * 16, 16)] = (
                        rowsA[i * 16 + jj, pl.ds(cc * 16, 16)] * cj)
            return 0
        lax.fori_loop(0, K // 16, _sca, 0)
        pltpu.async_copy(rowsA, accum.at[sidxA], ssemA, add=True)

        ccb.wait()
        ccb2.wait()

        def _cfb(i, _):
            w = ewB[pl.ds(i * 16, 16)]
            cv = cvalsB[pl.ds(i * 16, 16)] + cvals2B[pl.ds(i * 16, 16)]
            coefB[pl.ds(i * 16, 16)] = w / jnp.maximum(cv, 1.0)
            return 0
        lax.fori_loop(0, K // 16, _cfb, 0)
        gcb.wait()

        def _scb(i, _):
            cv = coefB[pl.ds(i * 16, 16)]
            for jj in range(16):
                cj = jnp.full((16,), cv[jj], jnp.float32)
                for cc in range(HALF // 16):
                    rowsB[i * 16 + jj, pl.ds(cc * 16, 16)] = (
                        rowsB[i * 16 + jj, pl.ds(cc * 16, 16)] * cj)
            return 0
        lax.fori_loop(0, K // 16, _scb, 0)
        pltpu.async_copy(rowsB, accum.at[sidxB], ssemB, add=True)
        return 0
    lax.fori_loop(0, ITERS // 2, _pair, 0)
    pltpu.make_async_copy(rowsA, accum.at[sidxA], ssemA).wait()
    pltpu.make_async_copy(rowsB, accum.at[sidxB], ssemB).wait()

    plsc.subcore_barrier()

    def _opair(it2, _):
        ooA = s * OPT + it2 * 2 * KO
        ooB = ooA + KO
        pltpu.async_copy(ids_hbm.at[pl.ds(ooA, KO)], oidxA, lsemA)
        pltpu.async_copy(ids_hbm.at[pl.ds(ooB, KO)], oidxB, lsemB)

        @pl.when(it2 > 0)
        def _():
            pltpu.make_async_copy(
                rowsA, out_hbm.at[pl.ds(ooA, KO), pl.ds(c * HALF, HALF)],
                ssemA).wait()
        pltpu.make_async_copy(ids_hbm.at[pl.ds(ooA, KO)], oidxA, lsemA).wait()
        pltpu.async_copy(accum.at[oidxA], rowsA, gsemA).wait()

        def _rla(j, _):
            for cc in range(HALF // 16):
                rowsA[j, pl.ds(cc * 16, 16)] = jnp.maximum(
                    rowsA[j, pl.ds(cc * 16, 16)], 0.0)
            return 0
        lax.fori_loop(0, KO, _rla, 0)
        pltpu.async_copy(
            rowsA, out_hbm.at[pl.ds(ooA, KO), pl.ds(c * HALF, HALF)], ssemA)

        @pl.when(it2 > 0)
        def _():
            pltpu.make_async_copy(
                rowsB, out_hbm.at[pl.ds(ooB, KO), pl.ds(c * HALF, HALF)],
                ssemB).wait()
        pltpu.make_async_copy(ids_hbm.at[pl.ds(ooB, KO)], oidxB, lsemB).wait()
        pltpu.async_copy(accum.at[oidxB], rowsB, gsemB).wait()

        def _rlb(j, _):
            for cc in range(HALF // 16):
                rowsB[j, pl.ds(cc * 16, 16)] = jnp.maximum(
                    rowsB[j, pl.ds(cc * 16, 16)], 0.0)
            return 0
        lax.fori_loop(0, KO, _rlb, 0)
        pltpu.async_copy(
            rowsB, out_hbm.at[pl.ds(ooB, KO), pl.ds(c * HALF, HALF)], ssemB)
        return 0
    lax.fori_loop(0, OPAIRS, _opair, 0)
    pltpu.make_async_copy(
        rowsA, out_hbm.at[pl.ds(0, KO), pl.ds(c * HALF, HALF)], ssemA).wait()
    pltpu.make_async_copy(
        rowsB, out_hbm.at[pl.ds(0, KO), pl.ds(c * HALF, HALF)], ssemB).wait()


_sc_kernel = functools.partial(
    pl.kernel,
    out_type=jax.ShapeDtypeStruct((IDS, 2 * HALF), jnp.float32),
    mesh=plsc.VectorSubcoreMesh(core_axis_name="c", subcore_axis_name="s"),
    scratch_types=[
        pltpu.VMEM_SHARED((NP, HALF), jnp.float32),
        pltpu.VMEM((3 * K,), jnp.int32),
        pltpu.VMEM((3 * K,), jnp.int32),
        pltpu.VMEM((K,), jnp.float32),
        pltpu.VMEM((K,), jnp.float32),
        pltpu.VMEM((K, HALF), jnp.float32),
        pltpu.VMEM((K, HALF), jnp.float32),
        pltpu.VMEM((K,), jnp.int32),
        pltpu.VMEM((K,), jnp.int32),
        pltpu.VMEM((K,), jnp.int32),
        pltpu.VMEM((K,), jnp.int32),
        pltpu.VMEM((K,), jnp.int32),
        pltpu.VMEM((K,), jnp.int32),
        pltpu.VMEM((K,), jnp.int32),
        pltpu.VMEM((K,), jnp.int32),
        pltpu.VMEM((K,), jnp.float32),
        pltpu.VMEM((K,), jnp.float32),
        pltpu.VMEM((K,), jnp.float32),
        pltpu.VMEM((K,), jnp.float32),
        pltpu.VMEM((K,), jnp.float32),
        pltpu.VMEM((K,), jnp.float32),
        pltpu.VMEM((KO,), jnp.int32),
        pltpu.VMEM((KO,), jnp.int32),
        pltpu.SemaphoreType.DMA,
        pltpu.SemaphoreType.DMA,
        pltpu.SemaphoreType.DMA,
        pltpu.SemaphoreType.DMA,
        pltpu.SemaphoreType.DMA,
        pltpu.SemaphoreType.DMA,
        pltpu.SemaphoreType.DMA,
        pltpu.SemaphoreType.DMA,
    ],
)(_sc_body)


def kernel(x, edge_index, edge_type, edge_weight, ent_user_ids, ent_item_ids,
           aspect_ent_ids, W_rel, W_root, bias):
    x = x.astype(jnp.float32)
    xp = jnp.pad(x, ((0, NP - N), (0, 0)))

    nb = W_rel.shape[1]
    eye = jnp.eye(nb, dtype=jnp.float32)
    w_bd = jnp.einsum('rbio,bc->rbico', W_rel.astype(jnp.float32), eye)
    w_bd = w_bd.reshape(R, D, D).transpose(1, 0, 2).reshape(D, R * D)
    w_cat = jnp.concatenate([w_bd, W_root.astype(jnp.float32)], axis=1)
    w_cat = w_cat.reshape(D, NREL, 2, HALF).transpose(0, 2, 1, 3).reshape(
        D, NREL * D)
    b_cat = jnp.concatenate(
        [jnp.zeros((R * D,), jnp.float32), bias.astype(jnp.float32)]
    ).reshape(1, NREL, 2, HALF).transpose(0, 2, 1, 3).reshape(1, NREL * D)

    table = _build_table(xp, w_cat, b_cat)

    agg = edge_index[0].astype(jnp.int32)
    msg = edge_index[1].astype(jnp.int32)
    typ = edge_type.astype(jnp.int32)
    ew = edge_weight.astype(jnp.float32)
    pad = EP - E
    epk = jnp.stack([
        jnp.pad(typ, (0, pad)),
        jnp.pad(msg, (0, pad)),
        jnp.pad(agg, (0, pad), constant_values=NP - 1),
    ])
    epk = epk.reshape(3, EP // K, K).transpose(1, 0, 2).reshape(-1)
    ewp = jnp.pad(ew, (0, pad))

    ids = jnp.concatenate(
        [ent_user_ids[:, None], ent_item_ids[:, None], aspect_ent_ids],
        axis=1).reshape(-1).astype(jnp.int32)

    cnt2 = _cnt_kernel(epk)
    outg = _sc_kernel(table, epk, ewp, ids, cnt2)
    return outg.reshape(B, 2 + ASP, D)


# speedup vs baseline: 1.0685x; 1.0685x over previous
import functools

import jax
import jax.numpy as jnp
from jax import lax
from jax.experimental import pallas as pl
from jax.experimental.pallas import tpu as pltpu
from jax.experimental.pallas import tpu_sc as plsc

N = 10002
NP = 10240
D = 256
E = 160000
R = 8
B = 1024
ASP = 30
HALF = 128
NREL = R + 1
IDS = B * (2 + ASP)
NSUB = 16
K = 128
EP = 163840
EPT = EP // NSUB
ITERS = EPT // K
KO = 128
OPT = IDS // NSUB
OPAIRS = OPT // KO // 2
ZB = 1024
BM = 1024
GN = NP // BM


def _mm_body(x_ref, w_ref, b_ref, o_ref):
    acc = jnp.dot(x_ref[...].astype(jnp.bfloat16),
                  w_ref[...].astype(jnp.bfloat16),
                  preferred_element_type=jnp.float32) + b_ref[...]
    o_ref[...] = acc.reshape(BM // 8, 8, 2 * NREL, HALF).transpose(
        0, 2, 1, 3).reshape(BM * 2 * NREL, HALF)


def _build_table(xp, w_cat, b_cat):
    y = pl.pallas_call(
        _mm_body,
        grid=(GN,),
        in_specs=[
            pl.BlockSpec((BM, D), lambda i: (i, 0)),
            pl.BlockSpec((D, NREL * D), lambda i: (0, 0)),
            pl.BlockSpec((1, NREL * D), lambda i: (0, 0)),
        ],
        out_specs=pl.BlockSpec((BM * 2 * NREL, HALF), lambda i: (i, 0)),
        out_shape=jax.ShapeDtypeStruct((NP * 2 * NREL, HALF), jnp.float32),
    )(xp, w_cat, b_cat)
    return y


def _fields(eb, i):
    t = eb[pl.ds(i * 16, 16)]
    m = eb[pl.ds(K + i * 16, 16)]
    a = eb[pl.ds(2 * K + i * 16, 16)]
    return t, m, a


def _cnt_body(epk_hbm, cnt_hbm, cnt, ebA, ebB, sidxA, sidxB, onesb, zb,
              lsemA, lsemB, ssemA, ssemB):
    s = lax.axis_index("s")

    def _z16(i, _):
        zb[pl.ds(i * 16, 16)] = jnp.zeros((16,), jnp.float32)
        return 0
    lax.fori_loop(0, ZB // 16, _z16, 0)
    for q in range(R * NP // NSUB // ZB):
        pltpu.sync_copy(zb, cnt.at[pl.ds(s * (R * NP // NSUB) + q * ZB, ZB)])
    for i in range(K // 16):
        onesb[pl.ds(i * 16, 16)] = jnp.ones((16,), jnp.float32)

    plsc.subcore_barrier()

    c = lax.axis_index("c")
    hw = ITERS // 4
    def _cpair(it2h, _):
        it2 = c * hw + it2h
        g0 = (s * ITERS + it2 * 2) * (3 * K)
        lda = pltpu.async_copy(epk_hbm.at[pl.ds(g0, 3 * K)], ebA, lsemA)
        ldb = pltpu.async_copy(epk_hbm.at[pl.ds(g0 + 3 * K, 3 * K)], ebB, lsemB)

        @pl.when(it2h > 0)
        def _():
            pltpu.make_async_copy(onesb, cnt.at[sidxA], ssemA).wait()
        lda.wait()

        def _mka(i, _):
            t, m, a = _fields(ebA, i)
            sidxA[pl.ds(i * 16, 16)] = t * NP + a
            return 0
        lax.fori_loop(0, K // 16, _mka, 0)
        pltpu.async_copy(onesb, cnt.at[sidxA], ssemA, add=True)

        @pl.when(it2h > 0)
        def _():
            pltpu.make_async_copy(onesb, cnt.at[sidxB], ssemB).wait()
        ldb.wait()

        def _mkb(i, _):
            t, m, a = _fields(ebB, i)
            sidxB[pl.ds(i * 16, 16)] = t * NP + a
            return 0
        lax.fori_loop(0, K // 16, _mkb, 0)
        pltpu.async_copy(onesb, cnt.at[sidxB], ssemB, add=True)
        return 0
    lax.fori_loop(0, hw, _cpair, 0)
    pltpu.make_async_copy(onesb, cnt.at[sidxA], ssemA).wait()
    pltpu.make_async_copy(onesb, cnt.at[sidxB], ssemB).wait()

    plsc.subcore_barrier()
    span = R * NP // NSUB
    o0 = s * span
    pltpu.sync_copy(cnt.at[pl.ds(o0, span)], cnt_hbm.at[pl.ds(c * R * NP + o0, span)])


_cnt_kernel = functools.partial(
    pl.kernel,
    out_type=jax.ShapeDtypeStruct((2 * R * NP,), jnp.float32),
    mesh=plsc.VectorSubcoreMesh(core_axis_name="c", subcore_axis_name="s"),
    scratch_types=[
        pltpu.VMEM_SHARED((R * NP,), jnp.float32),
        pltpu.VMEM((3 * K,), jnp.int32),
        pltpu.VMEM((3 * K,), jnp.int32),
        pltpu.VMEM((K,), jnp.int32),
        pltpu.VMEM((K,), jnp.int32),
        pltpu.VMEM((K,), jnp.float32),
        pltpu.VMEM((ZB,), jnp.float32),
        pltpu.SemaphoreType.DMA,
        pltpu.SemaphoreType.DMA,
        pltpu.SemaphoreType.DMA,
        pltpu.SemaphoreType.DMA,
    ],
)(_cnt_body)


def _sc_body(tbl_hbm, epk_hbm, ewp_hbm, ids_hbm, cnt_hbm, out_hbm,
             accum,
             ebA, ebB, ewA, ewB, rowsA, rowsB, gidxA, gidxB, cidxA, cidxB,
             sidxA, sidxB, cvalsA, cvalsB,
             coefA, coefB, oidxA, oidxB,
             lsemA, lsemB, gsemA, gsemB, csemA, csemB, ssemA, ssemB):
    c = lax.axis_index("c")
    s = lax.axis_index("s")

    lane16 = lax.iota(jnp.int32, 16)
    for q in range(NP // NSUB // K):
        n0 = s * (NP // NSUB) + q * K

        def _ri(i, _):
            n = n0 + i * 16 + lane16
            gidxA[pl.ds(i * 16, 16)] = (
                (n >> 3) * 144 + (16 + c) * 8 + (n & 7))
            return 0
        lax.fori_loop(0, K // 16, _ri, 0)
        pltpu.async_copy(tbl_hbm.at[gidxA], rowsA, gsemA).wait()
        pltpu.sync_copy(rowsA, accum.at[pl.ds(n0, K)])

    plsc.subcore_barrier()

    def _pair(it2, _):
        g0 = (s * ITERS + it2 * 2) * (3 * K)
        e0 = (s * ITERS + it2 * 2) * K
        lda = pltpu.async_copy(epk_hbm.at[pl.ds(g0, 3 * K)], ebA, lsemA)
        ldb = pltpu.async_copy(epk_hbm.at[pl.ds(g0 + 3 * K, 3 * K)], ebB, lsemB)
        lwa = pltpu.async_copy(ewp_hbm.at[pl.ds(e0, K)], ewA, gsemA)
        lwb = pltpu.async_copy(ewp_hbm.at[pl.ds(e0 + K, K)], ewB, gsemB)

        @pl.when(it2 > 0)
        def _():
            pltpu.make_async_copy(rowsA, accum.at[sidxA], ssemA).wait()
        lda.wait()
        lwa.wait()

        def _mka(i, _):
            t, m, a = _fields(ebA, i)
            gidxA[pl.ds(i * 16, 16)] = (
                (m >> 3) * 144 + (t * 2 + c) * 8 + (m & 7))
            cidxA[pl.ds(i * 16, 16)] = t * NP + a
            sidxA[pl.ds(i * 16, 16)] = a
            return 0
        lax.fori_loop(0, K // 16, _mka, 0)
        gca = pltpu.async_copy(tbl_hbm.at[gidxA], rowsA, gsemA)
        cca = pltpu.async_copy(cnt_hbm.at[cidxA], cvalsA, csemA)

        @pl.when(it2 > 0)
        def _():
            pltpu.make_async_copy(rowsB, accum.at[sidxB], ssemB).wait()
        ldb.wait()
        lwb.wait()

        def _mkb(i, _):
            t, m, a = _fields(ebB, i)
            gidxB[pl.ds(i * 16, 16)] = (
                (m >> 3) * 144 + (t * 2 + c) * 8 + (m & 7))
            cidxB[pl.ds(i * 16, 16)] = t * NP + a
            sidxB[pl.ds(i * 16, 16)] = a
            return 0
        lax.fori_loop(0, K // 16, _mkb, 0)
        gcb = pltpu.async_copy(tbl_hbm.at[gidxB], rowsB, gsemB)
        ccb = pltpu.async_copy(cnt_hbm.at[cidxB], cvalsB, csemB)

        cca.wait()

        def _cfa(i, _):
            w = ewA[pl.ds(i * 16, 16)]
            cv = cvalsA[pl.ds(i * 16, 16)]
            coefA[pl.ds(i * 16, 16)] = w / jnp.maximum(cv, 1.0)
            return 0
        lax.fori_loop(0, K // 16, _cfa, 0)
        gca.wait()

        def _sca(i, _):
            cv = coefA[pl.ds(i * 16, 16)]
            for jj in range(16):
                cj = jnp.full((16,), cv[jj], jnp.float32)
                for cc in range(HALF // 16):
                    rowsA[i * 16 + jj, pl.ds(cc * 16, 16)] = (
                        rowsA[i * 16 + jj, pl.ds(cc * 16, 16)] * cj)
            return 0
        lax.fori_loop(0, K // 16, _sca, 0)
        pltpu.async_copy(rowsA, accum.at[sidxA], ssemA, add=True)

        ccb.wait()

        def _cfb(i, _):
            w = ewB[pl.ds(i * 16, 16)]
            cv = cvalsB[pl.ds(i * 16, 16)]
            coefB[pl.ds(i * 16, 16)] = w / jnp.maximum(cv, 1.0)
            return 0
        lax.fori_loop(0, K // 16, _cfb, 0)
        gcb.wait()

        def _scb(i, _):
            cv = coefB[pl.ds(i * 16, 16)]
            for jj in range(16):
                cj = jnp.full((16,), cv[jj], jnp.float32)
                for cc in range(HALF // 16):
                    rowsB[i * 16 + jj, pl.ds(cc * 16, 16)] = (
                        rowsB[i * 16 + jj, pl.ds(cc * 16, 16)] * cj)
            return 0
        lax.fori_loop(0, K // 16, _scb, 0)
        pltpu.async_copy(rowsB, accum.at[sidxB], ssemB, add=True)
        return 0
    lax.fori_loop(0, ITERS // 2, _pair, 0)
    pltpu.make_async_copy(rowsA, accum.at[sidxA], ssemA).wait()
    pltpu.make_async_copy(rowsB, accum.at[sidxB], ssemB).wait()

    plsc.subcore_barrier()

    def _opair(it2, _):
        ooA = s * OPT + it2 * 2 * KO
        ooB = ooA + KO
        pltpu.async_copy(ids_hbm.at[pl.ds(ooA, KO)], oidxA, lsemA)
        pltpu.async_copy(ids_hbm.at[pl.ds(ooB, KO)], oidxB, lsemB)

        @pl.when(it2 > 0)
        def _():
            pltpu.make_async_copy(
                rowsA, out_hbm.at[pl.ds(ooA, KO), pl.ds(c * HALF, HALF)],
                ssemA).wait()
        pltpu.make_async_copy(ids_hbm.at[pl.ds(ooA, KO)], oidxA, lsemA).wait()
        pltpu.async_copy(accum.at[oidxA], rowsA, gsemA).wait()

        def _rla(j, _):
            for cc in range(HALF // 16):
                rowsA[j, pl.ds(cc * 16, 16)] = jnp.maximum(
                    rowsA[j, pl.ds(cc * 16, 16)], 0.0)
            return 0
        lax.fori_loop(0, KO, _rla, 0)
        pltpu.async_copy(
            rowsA, out_hbm.at[pl.ds(ooA, KO), pl.ds(c * HALF, HALF)], ssemA)

        @pl.when(it2 > 0)
        def _():
            pltpu.make_async_copy(
                rowsB, out_hbm.at[pl.ds(ooB, KO), pl.ds(c * HALF, HALF)],
                ssemB).wait()
        pltpu.make_async_copy(ids_hbm.at[pl.ds(ooB, KO)], oidxB, lsemB).wait()
        pltpu.async_copy(accum.at[oidxB], rowsB, gsemB).wait()

        def _rlb(j, _):
            for cc in range(HALF // 16):
                rowsB[j, pl.ds(cc * 16, 16)] = jnp.maximum(
                    rowsB[j, pl.ds(cc * 16, 16)], 0.0)
            return 0
        lax.fori_loop(0, KO, _rlb, 0)
        pltpu.async_copy(
            rowsB, out_hbm.at[pl.ds(ooB, KO), pl.ds(c * HALF, HALF)], ssemB)
        return 0
    lax.fori_loop(0, OPAIRS, _opair, 0)
    pltpu.make_async_copy(
        rowsA, out_hbm.at[pl.ds(0, KO), pl.ds(c * HALF, HALF)], ssemA).wait()
    pltpu.make_async_copy(
        rowsB, out_hbm.at[pl.ds(0, KO), pl.ds(c * HALF, HALF)], ssemB).wait()


_sc_kernel = functools.partial(
    pl.kernel,
    out_type=jax.ShapeDtypeStruct((IDS, 2 * HALF), jnp.float32),
    mesh=plsc.VectorSubcoreMesh(core_axis_name="c", subcore_axis_name="s"),
    scratch_types=[
        pltpu.VMEM_SHARED((NP, HALF), jnp.float32),
        pltpu.VMEM((3 * K,), jnp.int32),
        pltpu.VMEM((3 * K,), jnp.int32),
        pltpu.VMEM((K,), jnp.float32),
        pltpu.VMEM((K,), jnp.float32),
        pltpu.VMEM((K, HALF), jnp.float32),
        pltpu.VMEM((K, HALF), jnp.float32),
        pltpu.VMEM((K,), jnp.int32),
        pltpu.VMEM((K,), jnp.int32),
        pltpu.VMEM((K,), jnp.int32),
        pltpu.VMEM((K,), jnp.int32),
        pltpu.VMEM((K,), jnp.int32),
        pltpu.VMEM((K,), jnp.int32),
        pltpu.VMEM((K,), jnp.float32),
        pltpu.VMEM((K,), jnp.float32),
        pltpu.VMEM((K,), jnp.float32),
        pltpu.VMEM((K,), jnp.float32),
        pltpu.VMEM((KO,), jnp.int32),
        pltpu.VMEM((KO,), jnp.int32),
        pltpu.SemaphoreType.DMA,
        pltpu.SemaphoreType.DMA,
        pltpu.SemaphoreType.DMA,
        pltpu.SemaphoreType.DMA,
        pltpu.SemaphoreType.DMA,
        pltpu.SemaphoreType.DMA,
        pltpu.SemaphoreType.DMA,
        pltpu.SemaphoreType.DMA,
    ],
)(_sc_body)


def kernel(x, edge_index, edge_type, edge_weight, ent_user_ids, ent_item_ids,
           aspect_ent_ids, W_rel, W_root, bias):
    x = x.astype(jnp.float32)
    xp = jnp.pad(x, ((0, NP - N), (0, 0)))

    nb = W_rel.shape[1]
    eye = jnp.eye(nb, dtype=jnp.float32)
    w_bd = jnp.einsum('rbio,bc->rbico', W_rel.astype(jnp.float32), eye)
    w_bd = w_bd.reshape(R, D, D).transpose(1, 0, 2).reshape(D, R * D)
    w_cat = jnp.concatenate([w_bd, W_root.astype(jnp.float32)], axis=1)
    b_cat = jnp.concatenate(
        [jnp.zeros((R * D,), jnp.float32), bias.astype(jnp.float32)]
    ).reshape(1, NREL * D)

    table = _build_table(xp, w_cat, b_cat)

    agg = edge_index[0].astype(jnp.int32)
    msg = edge_index[1].astype(jnp.int32)
    typ = edge_type.astype(jnp.int32)
    ew = edge_weight.astype(jnp.float32)
    pad = EP - E
    epk = jnp.stack([
        jnp.pad(typ, (0, pad)),
        jnp.pad(msg, (0, pad)),
        jnp.pad(agg, (0, pad), constant_values=NP - 1),
    ])
    epk = epk.reshape(3, EP // K, K).transpose(1, 0, 2).reshape(-1)
    ewp = jnp.pad(ew, (0, pad))

    ids = jnp.concatenate(
        [ent_user_ids[:, None], ent_item_ids[:, None], aspect_ent_ids],
        axis=1).reshape(-1).astype(jnp.int32)

    cnt2 = _cnt_kernel(epk)
    cnt1 = cnt2[:R * NP] + cnt2[R * NP:]
    outg = _sc_kernel(table, epk, ewp, ids, cnt1)
    return outg.reshape(B, 2 + ASP, D)
